# baseline (speedup 1.0000x reference)
import functools

import jax
import jax.numpy as jnp
from jax import lax
from jax.experimental import pallas as pl
from jax.experimental.pallas import tpu as pltpu

_ROWS = 16384
_DIM = 2048
_EXPERTS = 64
_TILE = 1024


def _router_kernel(x_ref, wt_ref, b_ref, out_ref):
    logits = lax.dot_general(wt_ref[...], x_ref[...],
                             (((1,), (1,)), ((), ())),
                             preferred_element_type=jnp.float32)
    logits = logits + b_ref[...]
    t = logits.shape[1]
    iota = jax.lax.broadcasted_iota(
        jnp.int32, (_EXPERTS, t), 0).astype(jnp.float32)
    m1 = jnp.max(logits, axis=0, keepdims=True)
    i1 = jnp.min(jnp.where(logits == m1, iota, float(_EXPERTS)),
                 axis=0, keepdims=True)
    masked = jnp.where(iota == i1, -jnp.inf, logits)
    m2 = jnp.max(masked, axis=0, keepdims=True)
    i2 = jnp.min(jnp.where(masked == m2, iota, float(_EXPERTS)),
                 axis=0, keepdims=True)
    out_ref[...] = jnp.concatenate([i1, i2], axis=0).astype(jnp.int32).T


@jax.jit
def kernel(x, W, b):
    wt = W.T
    b2 = b.reshape(_EXPERTS, 1)
    grid = (_ROWS // _TILE,)
    out_t = pl.pallas_call(
        _router_kernel,
        grid=grid,
        in_specs=[
            pl.BlockSpec((_TILE, _DIM), lambda i: (i, 0)),
            pl.BlockSpec((_EXPERTS, _DIM), lambda i: (0, 0)),
            pl.BlockSpec((_EXPERTS, 1), lambda i: (0, 0)),
        ],
        out_specs=pl.BlockSpec((_TILE, 2), lambda i: (i, 0)),
        out_shape=jax.ShapeDtypeStruct((_ROWS, 2), jnp.int32),
        compiler_params=pltpu.CompilerParams(
            dimension_semantics=("parallel",),
        ),
    )(x, wt, b2)
    return out_t


# speedup vs baseline: 1.0096x; 1.0096x over previous
import jax
import jax.numpy as jnp
from jax import lax
from jax.experimental import pallas as pl
from jax.experimental.pallas import tpu as pltpu

_ROWS = 16384
_DIM = 2048
_EXPERTS = 64
_TILE = 2048


def _router_kernel(x_ref, wt_ref, b_ref, out_ref):
    logits = lax.dot_general(wt_ref[...], x_ref[...],
                             (((1,), (1,)), ((), ())),
                             preferred_element_type=jnp.float32)
    logits = logits + b_ref[...]
    t = logits.shape[1]
    iota = jax.lax.broadcasted_iota(
        jnp.int32, (_EXPERTS, t), 0).astype(jnp.float32)
    m1 = jnp.max(logits, axis=0, keepdims=True)
    i1 = jnp.min(jnp.where(logits == m1, iota, float(_EXPERTS)),
                 axis=0, keepdims=True)
    masked = jnp.where(iota == i1, -jnp.inf, logits)
    m2 = jnp.max(masked, axis=0, keepdims=True)
    i2 = jnp.min(jnp.where(masked == m2, iota, float(_EXPERTS)),
                 axis=0, keepdims=True)
    out_ref[...] = jnp.concatenate([i1, i2], axis=0).astype(jnp.int32).T


@jax.jit
def kernel(x, W, b):
    wt = W.T
    b2 = b.reshape(_EXPERTS, 1)
    grid = (_ROWS // _TILE,)
    out_t = pl.pallas_call(
        _router_kernel,
        grid=grid,
        in_specs=[
            pl.BlockSpec((_TILE, _DIM), lambda i: (i, 0)),
            pl.BlockSpec((_EXPERTS, _DIM), lambda i: (0, 0)),
            pl.BlockSpec((_EXPERTS, 1), lambda i: (0, 0)),
        ],
        out_specs=pl.BlockSpec((_TILE, 2), lambda i: (i, 0)),
        out_shape=jax.ShapeDtypeStruct((_ROWS, 2), jnp.int32),
        compiler_params=pltpu.CompilerParams(
            dimension_semantics=("parallel",),
        ),
    )(x, wt, b2)
    return out_t
